# baseline (speedup 1.0000x reference)
import functools

import jax
import jax.numpy as jnp
from jax import lax
from jax.experimental import pallas as pl
from jax.experimental.pallas import tpu as pltpu
from jax.experimental.pallas import tpu_sc as plsc

BATCH = 16384
DIM = 128
HALF = DIM // 2
LANES = 16
NC = 2
NS = 16
NW = NC * NS
ROWS_PER_W = BATCH // NW
CHUNK = 128
NCHUNK = ROWS_PER_W // CHUNK
NBUF = 2


def _body(emb_hbm, idx_hbm, real_hbm, imag_hbm, out_hbm,
          idx_v, emb_v0, emb_v1, re_v0, re_v1, im_v0, im_v1,
          out_v0, out_v1, in_sem0, in_sem1, out_sem0, out_sem1):
    wid = lax.axis_index("s") * NC + lax.axis_index("c")
    emb_v = (emb_v0, emb_v1)
    re_v = (re_v0, re_v1)
    im_v = (im_v0, im_v1)
    out_v = (out_v0, out_v1)
    in_sem = (in_sem0, in_sem1)
    out_sem = (out_sem0, out_sem1)

    pltpu.sync_copy(idx_hbm.at[pl.ds(wid * NCHUNK, NCHUNK)], idx_v)

    def start_in(c):
        b = c % NBUF
        base = wid * ROWS_PER_W + c * CHUNK
        return [
            pltpu.async_copy(emb_hbm.at[pl.ds(base, CHUNK)], emb_v[b], in_sem[b]),
            pltpu.async_copy(real_hbm.at[idx_v.at[c]], re_v[b], in_sem[b]),
            pltpu.async_copy(imag_hbm.at[idx_v.at[c]], im_v[b], in_sem[b]),
        ]

    pending = {0: start_in(0)}
    out_pending = {}
    for c in range(NCHUNK):
        b = c % NBUF
        if c + 1 < NCHUNK:
            pending[c + 1] = start_in(c + 1)
        for h in pending.pop(c):
            h.wait()
        if c - NBUF in out_pending:
            out_pending.pop(c - NBUF).wait()
        ev, rv, iv, ov = emb_v[b], re_v[b], im_v[b], out_v[b]

        def row(i, carry):
            for j in range(HALF // LANES):
                er = ev[i, pl.ds(j * LANES, LANES)]
                ei = ev[i, pl.ds(HALF + j * LANES, LANES)]
                rr = rv[i, pl.ds(j * LANES, LANES)]
                ri = iv[i, pl.ds(j * LANES, LANES)]
                ov[i, pl.ds(j * LANES, LANES)] = er * rr - ei * ri
                ov[i, pl.ds(HALF + j * LANES, LANES)] = er * ri + ei * rr
            return carry

        lax.fori_loop(0, CHUNK, row, 0)
        base = wid * ROWS_PER_W + c * CHUNK
        out_pending[c] = pltpu.async_copy(
            ov, out_hbm.at[pl.ds(base, CHUNK)], out_sem[b])
    for h in out_pending.values():
        h.wait()


_sc_call = functools.partial(
    pl.kernel,
    out_type=jax.ShapeDtypeStruct((BATCH, DIM), jnp.float32),
    mesh=plsc.VectorSubcoreMesh(core_axis_name="c", subcore_axis_name="s"),
    compiler_params=pltpu.CompilerParams(use_tc_tiling_on_sc=False),
    scratch_types=[
        pltpu.VMEM((NCHUNK, CHUNK), jnp.int32),
        pltpu.VMEM((CHUNK, DIM), jnp.float32),
        pltpu.VMEM((CHUNK, DIM), jnp.float32),
        pltpu.VMEM((CHUNK, HALF), jnp.float32),
        pltpu.VMEM((CHUNK, HALF), jnp.float32),
        pltpu.VMEM((CHUNK, HALF), jnp.float32),
        pltpu.VMEM((CHUNK, HALF), jnp.float32),
        pltpu.VMEM((CHUNK, DIM), jnp.float32),
        pltpu.VMEM((CHUNK, DIM), jnp.float32),
        pltpu.SemaphoreType.DMA,
        pltpu.SemaphoreType.DMA,
        pltpu.SemaphoreType.DMA,
        pltpu.SemaphoreType.DMA,
    ],
)(_body)


def kernel(emb, rel_index, real, imag):
    idx = rel_index.astype(jnp.int32).reshape(NW * NCHUNK, CHUNK)
    return _sc_call(emb, idx, real, imag)


# speedup vs baseline: 1.5690x; 1.5690x over previous
import functools

import jax
import jax.numpy as jnp
from jax import lax
from jax.experimental import pallas as pl
from jax.experimental.pallas import tpu as pltpu
from jax.experimental.pallas import tpu_sc as plsc

BATCH = 16384
DIM = 128
HALF = DIM // 2
LANES = 16
NC = 2
NS = 16
NW = NC * NS
ROWS_PER_W = BATCH // NW
CHUNK = 32
NCHUNK = ROWS_PER_W // CHUNK
NBUF = 2


def _body(emb_hbm, idx_hbm, real_hbm, imag_hbm, out_hbm,
          idx_v,
          emb_v0, emb_v1, gr_v0, gr_v1, gi_v0, gi_v1, out_v0, out_v1,
          in_sem0, in_sem1, out_sem0, out_sem1):
    wid = lax.axis_index("s") * NC + lax.axis_index("c")
    emb_v = (emb_v0, emb_v1)
    gr_v = (gr_v0, gr_v1)
    gi_v = (gi_v0, gi_v1)
    out_v = (out_v0, out_v1)
    in_sem = (in_sem0, in_sem1)
    out_sem = (out_sem0, out_sem1)

    pltpu.sync_copy(idx_hbm.at[pl.ds(wid * ROWS_PER_W, ROWS_PER_W)], idx_v)

    def start_in(c):
        b = c % NBUF
        base = wid * ROWS_PER_W + c * CHUNK
        handles = [
            pltpu.async_copy(emb_hbm.at[pl.ds(base, CHUNK)], emb_v[b],
                             in_sem[b]),
        ]
        for g in range(CHUNK // LANES):
            v = idx_v[pl.ds(c * CHUNK + g * LANES, LANES)]
            for k in range(LANES):
                r = v[k]
                i = g * LANES + k
                handles.append(pltpu.async_copy(
                    real_hbm.at[r], gr_v[b].at[i], in_sem[b]))
                handles.append(pltpu.async_copy(
                    imag_hbm.at[r], gi_v[b].at[i], in_sem[b]))
        return handles

    pending = {0: start_in(0)}
    out_pending = {}
    for c in range(NCHUNK):
        b = c % NBUF
        if c + 1 < NCHUNK:
            pending[c + 1] = start_in(c + 1)
        for h in pending.pop(c):
            h.wait()
        if c - NBUF in out_pending:
            out_pending.pop(c - NBUF).wait()
        ev, rv, iv, ov = emb_v[b], gr_v[b], gi_v[b], out_v[b]

        def row(i, carry):
            for j in range(HALF // LANES):
                er = ev[i, pl.ds(j * LANES, LANES)]
                ei = ev[i, pl.ds(HALF + j * LANES, LANES)]
                rr = rv[i, pl.ds(j * LANES, LANES)]
                ri = iv[i, pl.ds(j * LANES, LANES)]
                ov[i, pl.ds(j * LANES, LANES)] = er * rr - ei * ri
                ov[i, pl.ds(HALF + j * LANES, LANES)] = er * ri + ei * rr
            return carry

        lax.fori_loop(0, CHUNK, row, 0)
        base = wid * ROWS_PER_W + c * CHUNK
        out_pending[c] = pltpu.async_copy(
            ov, out_hbm.at[pl.ds(base, CHUNK)], out_sem[b])
    for h in out_pending.values():
        h.wait()


_sc_call = functools.partial(
    pl.kernel,
    out_type=jax.ShapeDtypeStruct((BATCH, DIM), jnp.float32),
    mesh=plsc.VectorSubcoreMesh(core_axis_name="c", subcore_axis_name="s"),
    scratch_types=[
        pltpu.VMEM((ROWS_PER_W,), jnp.int32),
        pltpu.VMEM((CHUNK, DIM), jnp.float32),
        pltpu.VMEM((CHUNK, DIM), jnp.float32),
        pltpu.VMEM((CHUNK, HALF), jnp.float32),
        pltpu.VMEM((CHUNK, HALF), jnp.float32),
        pltpu.VMEM((CHUNK, HALF), jnp.float32),
        pltpu.VMEM((CHUNK, HALF), jnp.float32),
        pltpu.VMEM((CHUNK, DIM), jnp.float32),
        pltpu.VMEM((CHUNK, DIM), jnp.float32),
        pltpu.SemaphoreType.DMA,
        pltpu.SemaphoreType.DMA,
        pltpu.SemaphoreType.DMA,
        pltpu.SemaphoreType.DMA,
    ],
)(_body)


def kernel(emb, rel_index, real, imag):
    return _sc_call(emb, rel_index.astype(jnp.int32), real, imag)


# speedup vs baseline: 1.5840x; 1.0096x over previous
import functools

import jax
import jax.numpy as jnp
from jax import lax
from jax.experimental import pallas as pl
from jax.experimental.pallas import tpu as pltpu
from jax.experimental.pallas import tpu_sc as plsc

BATCH = 16384
DIM = 128
HALF = DIM // 2
LANES = 16
NC = 2
NS = 16
NW = NC * NS
ROWS_PER_W = BATCH // NW
CHUNK = 64
NCHUNK = ROWS_PER_W // CHUNK
NBUF = 2


def _body(emb_hbm, idx_hbm, real_hbm, imag_hbm, out_hbm,
          idx_v,
          emb_v0, emb_v1, gr_v0, gr_v1, gi_v0, gi_v1, out_v0, out_v1,
          in_sem0, in_sem1, out_sem0, out_sem1):
    wid = lax.axis_index("s") * NC + lax.axis_index("c")
    emb_v = (emb_v0, emb_v1)
    gr_v = (gr_v0, gr_v1)
    gi_v = (gi_v0, gi_v1)
    out_v = (out_v0, out_v1)
    in_sem = (in_sem0, in_sem1)
    out_sem = (out_sem0, out_sem1)

    pltpu.sync_copy(idx_hbm.at[pl.ds(wid * ROWS_PER_W, ROWS_PER_W)], idx_v)

    def start_in(c):
        b = c % NBUF
        base = wid * ROWS_PER_W + c * CHUNK
        pltpu.async_copy(emb_hbm.at[pl.ds(base, CHUNK)], emb_v[b], in_sem[b])
        for g in range(CHUNK // LANES):
            v = idx_v[pl.ds(c * CHUNK + g * LANES, LANES)]
            for k in range(LANES):
                r = v[k]
                i = g * LANES + k
                pltpu.async_copy(real_hbm.at[r], gr_v[b].at[i], in_sem[b])
                pltpu.async_copy(imag_hbm.at[r], gi_v[b].at[i], in_sem[b])

    def drain_in(b):
        pltpu.make_async_copy(
            emb_hbm.at[pl.ds(0, CHUNK)], emb_v[b], in_sem[b]).wait()
        pltpu.make_async_copy(
            real_hbm.at[pl.ds(0, CHUNK)], gr_v[b], in_sem[b]).wait()
        pltpu.make_async_copy(
            imag_hbm.at[pl.ds(0, CHUNK)], gi_v[b], in_sem[b]).wait()

    start_in(0)
    out_pending = {}
    for c in range(NCHUNK):
        b = c % NBUF
        if c + 1 < NCHUNK:
            start_in(c + 1)
        drain_in(b)
        if c - NBUF in out_pending:
            out_pending.pop(c - NBUF).wait()
        ev, rv, iv, ov = emb_v[b], gr_v[b], gi_v[b], out_v[b]

        def row(i, carry):
            for j in range(HALF // LANES):
                er = ev[i, pl.ds(j * LANES, LANES)]
                ei = ev[i, pl.ds(HALF + j * LANES, LANES)]
                rr = rv[i, pl.ds(j * LANES, LANES)]
                ri = iv[i, pl.ds(j * LANES, LANES)]
                ov[i, pl.ds(j * LANES, LANES)] = er * rr - ei * ri
                ov[i, pl.ds(HALF + j * LANES, LANES)] = er * ri + ei * rr
            return carry

        lax.fori_loop(0, CHUNK, row, 0)
        base = wid * ROWS_PER_W + c * CHUNK
        out_pending[c] = pltpu.async_copy(
            ov, out_hbm.at[pl.ds(base, CHUNK)], out_sem[b])
    for h in out_pending.values():
        h.wait()


_sc_call = functools.partial(
    pl.kernel,
    out_type=jax.ShapeDtypeStruct((BATCH, DIM), jnp.float32),
    mesh=plsc.VectorSubcoreMesh(core_axis_name="c", subcore_axis_name="s"),
    scratch_types=[
        pltpu.VMEM((ROWS_PER_W,), jnp.int32),
        pltpu.VMEM((CHUNK, DIM), jnp.float32),
        pltpu.VMEM((CHUNK, DIM), jnp.float32),
        pltpu.VMEM((CHUNK, HALF), jnp.float32),
        pltpu.VMEM((CHUNK, HALF), jnp.float32),
        pltpu.VMEM((CHUNK, HALF), jnp.float32),
        pltpu.VMEM((CHUNK, HALF), jnp.float32),
        pltpu.VMEM((CHUNK, DIM), jnp.float32),
        pltpu.VMEM((CHUNK, DIM), jnp.float32),
        pltpu.SemaphoreType.DMA,
        pltpu.SemaphoreType.DMA,
        pltpu.SemaphoreType.DMA,
        pltpu.SemaphoreType.DMA,
    ],
)(_body)


def kernel(emb, rel_index, real, imag):
    return _sc_call(emb, rel_index.astype(jnp.int32), real, imag)
